# baseline (speedup 1.0000x reference)
import functools

import jax
import jax.numpy as jnp
from jax import lax
from jax.experimental import pallas as pl
from jax.experimental.pallas import tpu as pltpu
from jax.experimental.pallas import tpu_sc as plsc

_NUM_CORES = 2
_NUM_SUBCORES = 16
_NSHIFT = 16
_PLANE = 4096
_BPW = 64
_WIN = 16 * (_BPW - 1) + 2048


@functools.partial(jax.jit, static_argnums=(1, 2))
def _expand_bias(rev16, H, S):
    mesh = plsc.VectorSubcoreMesh(core_axis_name="c", subcore_axis_name="s")

    @functools.partial(
        pl.kernel,
        out_type=jax.ShapeDtypeStruct((H, S, S), jnp.float32),
        mesh=mesh,
        scratch_types=[
            pltpu.VMEM_SHARED((_NUM_SUBCORES, H, _WIN), jnp.float32),
            pltpu.SemaphoreType.DMA,
            pltpu.SemaphoreType.DMA,
        ],
        compiler_params=pltpu.CompilerParams(use_tc_tiling_on_sc=False),
    )
    def body(rev_hbm, out_hbm, shared, gsem, ssem):
        sid = lax.axis_index("s")
        wid = sid * _NUM_CORES + lax.axis_index("c")
        rho = wid % _NSHIFT
        half = wid // _NSHIFT
        b0 = half * _BPW
        s = (_NSHIFT - 1) - rho
        qmin = (S // _NSHIFT) - b0 - _BPW
        buf = shared.at[sid]

        pltpu.make_async_copy(
            rev_hbm.at[s, :, pl.ds(qmin * _NSHIFT, _WIN)], buf, gsem
        ).start()
        pltpu.make_async_copy(
            rev_hbm.at[0, :, pl.ds(0, _WIN)], buf, gsem
        ).wait()

        def issue(t, carry):
            i = rho + _NSHIFT * (b0 + t)
            pltpu.make_async_copy(
                buf.at[:, pl.ds(_NSHIFT * (_BPW - 1 - t), S)],
                out_hbm.at[:, i, :],
                ssem,
            ).start()
            return carry

        lax.fori_loop(0, _BPW, issue, 0)
        pltpu.make_async_copy(
            out_hbm.at[:, pl.ds(0, _BPW), :],
            out_hbm.at[:, pl.ds(0, _BPW), :],
            ssem,
        ).wait()

    return body(rev16)


def kernel(seq_len, table):
    del seq_len
    R, H = table.shape
    S = (R + 1) // 2
    rev = table[::-1, :].T
    rev_pad = jnp.pad(rev, ((0, 0), (0, _PLANE + _NSHIFT - 1 - rev.shape[1])))
    rev16 = jnp.stack([rev_pad[:, s:s + _PLANE] for s in range(_NSHIFT)])
    rows = _expand_bias(rev16, H, S)
    return rows.reshape(1, H, S, S)
